# baseline (speedup 1.0000x reference)
import jax
import jax.numpy as jnp
from jax.experimental import pallas as pl
from jax.experimental.pallas import tpu as pltpu

_XT_W = (((0,), (1,)), ((), ()))
_HT_W = (((1,), (1,)), ((), ()))
_BN_SCALE = 1.0 / (1.0 + 1e-5) ** 0.5
_OUT_PAD = 32


def _fused_kernel(x_hbm, xyz_ref,
                  cW1, cb1, cg1, cbe1, cW2, cb2, cg2, cbe2,
                  sW1, sb1, sg1, sbe1, sW2, sb2, sg2, sbe2,
                  hW1, hb1, hg1, hbe1, hW2, hb2, hg2, hbe2,
                  mW1, mb1, mg1, mbe1, mW2, mb2, mg2, mbe2,
                  Wc, bc, Ws, bs, Wh, bh, W3, b3,
                  out_ref, xbuf, sem):
    B = x_hbm.shape[0]

    for b in range(B):
        pltpu.make_async_copy(x_hbm.at[b], xbuf.at[b], sem.at[b]).start()

    g1 = jnp.concatenate([cg1[...], sg1[...], hg1[...], mg1[...]], axis=1)
    b1 = jnp.concatenate([cb1[...], sb1[...], hb1[...], mb1[...]], axis=1)
    be1 = jnp.concatenate([cbe1[...], sbe1[...], hbe1[...], mbe1[...]], axis=1)
    s1 = g1 * _BN_SCALE
    W1 = jnp.concatenate(
        [cW1[...], sW1[...], hW1[...], mW1[...]], axis=0)
    W1 = (W1 * s1[0][:, None]).astype(jnp.bfloat16)
    bias1 = s1 * b1 + be1

    def prep2(W2, b2, g2, be2, col_lo):
        s = g2[...] * _BN_SCALE
        Wb = jnp.pad(W2[...] * s[0][:, None],
                     ((0, 0), (col_lo, 384 - col_lo)))
        return Wb.astype(jnp.bfloat16), s * b2[...] + be2[...]

    W2s = [prep2(cW2, cb2, cg2, cbe2, 0), prep2(sW2, sb2, sg2, sbe2, 128),
           prep2(hW2, hb2, hg2, hbe2, 256), prep2(mW2, mb2, mg2, mbe2, 384)]
    W2blk = jnp.concatenate([w for w, _ in W2s], axis=0)
    bias2 = jnp.concatenate([b for _, b in W2s], axis=1)

    def padw(W, row_lo, n_rows):
        return jnp.pad(W[...],
                       ((row_lo, _OUT_PAD - row_lo - n_rows), (0, 0))
                       ).astype(jnp.bfloat16)

    Wfcat = jnp.concatenate(
        [padw(Wc, 0, 3), padw(Ws, 3, 3), padw(Wh, 6, 2), padw(W3, 8, 20)],
        axis=1)
    bias_f = jnp.concatenate([bc[...], bs[...], bh[...], b3[...]], axis=1)
    bias_f = jnp.pad(bias_f, ((0, 0), (0, _OUT_PAD - 28)))

    h1s = []
    for b in range(B):
        pltpu.make_async_copy(x_hbm.at[b], xbuf.at[b], sem.at[b]).wait()
        x = xbuf[b].astype(jnp.bfloat16)
        h1 = jax.lax.dot_general(x, W1, _XT_W,
                                 preferred_element_type=jnp.float32)
        h1s.append(jnp.maximum(h1 + bias1, 0.0).astype(jnp.bfloat16))

    for b in range(B):
        h2 = jax.lax.dot_general(h1s[b], W2blk, _HT_W,
                                 preferred_element_type=jnp.float32)
        h2 = jnp.maximum(h2 + bias2, 0.0).astype(jnp.bfloat16)
        y = jax.lax.dot_general(h2, Wfcat, _HT_W,
                                preferred_element_type=jnp.float32)
        y = y + (jnp.pad(xyz_ref[b], ((0, 0), (0, _OUT_PAD - 3))) + bias_f)
        out_ref[b] = y[:, :28]


def kernel(vote_features, aggregated_vote_xyz, params):
    B, C, K = vote_features.shape

    def head_args(p):
        return [p['W1'], p['b1'][None, :], p['g1'][None, :], p['be1'][None, :],
                p['W2'], p['b2'][None, :], p['g2'][None, :], p['be2'][None, :]]

    g = params['gmm']
    ps = params['sem']
    args = ([vote_features, aggregated_vote_xyz]
            + head_args(params['center'])
            + head_args(params['size'])
            + head_args(params['heading'])
            + head_args(params['sem'])
            + [g['Wc'], g['bc'][None, :], g['Ws'], g['bs'][None, :],
               g['Wh'], g['bh'][None, :], ps['W3'], ps['b3'][None, :]])

    in_specs = ([pl.BlockSpec(memory_space=pltpu.MemorySpace.HBM),
                 pl.BlockSpec(memory_space=pltpu.MemorySpace.VMEM)]
                + [pl.BlockSpec(memory_space=pltpu.MemorySpace.VMEM)
                   for _ in args[2:]])

    out = pl.pallas_call(
        _fused_kernel,
        in_specs=in_specs,
        out_specs=pl.BlockSpec(memory_space=pltpu.MemorySpace.VMEM),
        out_shape=jax.ShapeDtypeStruct((B, K, 28), jnp.float32),
        scratch_shapes=[
            pltpu.VMEM((B, C, K), jnp.float32),
            pltpu.SemaphoreType.DMA((B,)),
        ],
    )(*args)
    return out


# speedup vs baseline: 1.0660x; 1.0660x over previous
import jax
import jax.numpy as jnp
from jax.experimental import pallas as pl
from jax.experimental.pallas import tpu as pltpu

_XT_W = (((0,), (1,)), ((), ()))
_HT_W = (((1,), (1,)), ((), ()))
_BN_SCALE = 1.0 / (1.0 + 1e-5) ** 0.5
_OUT_PAD = 32


def _fused_kernel(x_hbm, xyz_ref,
                  cW1, cb1, cg1, cbe1, cW2, cb2, cg2, cbe2,
                  sW1, sb1, sg1, sbe1, sW2, sb2, sg2, sbe2,
                  hW1, hb1, hg1, hbe1, hW2, hb2, hg2, hbe2,
                  mW1, mb1, mg1, mbe1, mW2, mb2, mg2, mbe2,
                  Wc, bc, Ws, bs, Wh, bh, W3, b3,
                  out_ref, xbuf, sem):
    B = x_hbm.shape[0]

    for b in range(B):
        pltpu.make_async_copy(x_hbm.at[b], xbuf.at[b], sem.at[b]).start()

    g1 = jnp.concatenate([cg1[...], sg1[...], hg1[...], mg1[...]], axis=1)
    b1 = jnp.concatenate([cb1[...], sb1[...], hb1[...], mb1[...]], axis=1)
    be1 = jnp.concatenate([cbe1[...], sbe1[...], hbe1[...], mbe1[...]], axis=1)
    s1 = g1 * _BN_SCALE
    W1 = jnp.concatenate(
        [cW1[...], sW1[...], hW1[...], mW1[...]], axis=0)
    W1 = (W1 * s1[0][:, None]).astype(jnp.bfloat16)
    bias1 = s1 * b1 + be1

    def prep2(W2, b2, g2, be2):
        s = g2[...] * _BN_SCALE
        return (W2[...] * s[0][:, None]).astype(jnp.bfloat16), \
            s * b2[...] + be2[...]

    W2s = [prep2(cW2, cb2, cg2, cbe2), prep2(sW2, sb2, sg2, sbe2),
           prep2(hW2, hb2, hg2, hbe2), prep2(mW2, mb2, mg2, mbe2)]

    def padw(W, row_lo, n_rows):
        return jnp.pad(W[...],
                       ((row_lo, _OUT_PAD - row_lo - n_rows), (0, 0))
                       ).astype(jnp.bfloat16)

    Wf = [padw(Wc, 0, 3), padw(Ws, 3, 3), padw(Wh, 6, 2), padw(W3, 8, 20)]
    bias_f = jnp.concatenate([bc[...], bs[...], bh[...], b3[...]], axis=1)
    bias_f = jnp.pad(bias_f, ((0, 0), (0, _OUT_PAD - 28)))

    h1s = []
    for b in range(B):
        pltpu.make_async_copy(x_hbm.at[b], xbuf.at[b], sem.at[b]).wait()
        x = xbuf[b].astype(jnp.bfloat16)
        h1 = jax.lax.dot_general(x, W1, _XT_W,
                                 preferred_element_type=jnp.float32)
        h1s.append(jnp.maximum(h1 + bias1, 0.0).astype(jnp.bfloat16))

    outs = [jnp.pad(xyz_ref[b], ((0, 0), (0, _OUT_PAD - 3))) + bias_f
            for b in range(B)]
    for i in range(4):
        W2, bias2 = W2s[i]
        for b in range(B):
            h2 = jax.lax.dot_general(h1s[b][:, i * 128:(i + 1) * 128], W2,
                                     _HT_W, preferred_element_type=jnp.float32)
            h2 = jnp.maximum(h2 + bias2, 0.0).astype(jnp.bfloat16)
            outs[b] = outs[b] + jax.lax.dot_general(
                h2, Wf[i], _HT_W, preferred_element_type=jnp.float32)
    for b in range(B):
        out_ref[b] = outs[b][:, :28]


def kernel(vote_features, aggregated_vote_xyz, params):
    B, C, K = vote_features.shape

    def head_args(p):
        return [p['W1'], p['b1'][None, :], p['g1'][None, :], p['be1'][None, :],
                p['W2'], p['b2'][None, :], p['g2'][None, :], p['be2'][None, :]]

    g = params['gmm']
    ps = params['sem']
    args = ([vote_features, aggregated_vote_xyz]
            + head_args(params['center'])
            + head_args(params['size'])
            + head_args(params['heading'])
            + head_args(params['sem'])
            + [g['Wc'], g['bc'][None, :], g['Ws'], g['bs'][None, :],
               g['Wh'], g['bh'][None, :], ps['W3'], ps['b3'][None, :]])

    in_specs = ([pl.BlockSpec(memory_space=pltpu.MemorySpace.HBM),
                 pl.BlockSpec(memory_space=pltpu.MemorySpace.VMEM)]
                + [pl.BlockSpec(memory_space=pltpu.MemorySpace.VMEM)
                   for _ in args[2:]])

    out = pl.pallas_call(
        _fused_kernel,
        in_specs=in_specs,
        out_specs=pl.BlockSpec(memory_space=pltpu.MemorySpace.VMEM),
        out_shape=jax.ShapeDtypeStruct((B, K, 28), jnp.float32),
        scratch_shapes=[
            pltpu.VMEM((B, C, K), jnp.float32),
            pltpu.SemaphoreType.DMA((B,)),
        ],
    )(*args)
    return out


# speedup vs baseline: 1.0949x; 1.0272x over previous
import jax
import jax.numpy as jnp
from jax.experimental import pallas as pl
from jax.experimental.pallas import tpu as pltpu

_XT_W = (((0,), (1,)), ((), ()))
_HT_W = (((1,), (1,)), ((), ()))
_BN_SCALE = 1.0 / (1.0 + 1e-5) ** 0.5
_OUT_PAD = 32


def _fused_kernel(x_hbm, xyz_ref,
                  cW1, cb1, cg1, cbe1, cW2, cb2, cg2, cbe2,
                  sW1, sb1, sg1, sbe1, sW2, sb2, sg2, sbe2,
                  hW1, hb1, hg1, hbe1, hW2, hb2, hg2, hbe2,
                  mW1, mb1, mg1, mbe1, mW2, mb2, mg2, mbe2,
                  Wc, bc, Ws, bs, Wh, bh, W3, b3,
                  out_ref, xbuf, obuf, sem, osem):
    B = x_hbm.shape[0]

    for b in range(B):
        pltpu.make_async_copy(x_hbm.at[b], xbuf.at[b], sem.at[b]).start()

    g1 = jnp.concatenate([cg1[...], sg1[...], hg1[...], mg1[...]], axis=1)
    b1 = jnp.concatenate([cb1[...], sb1[...], hb1[...], mb1[...]], axis=1)
    be1 = jnp.concatenate([cbe1[...], sbe1[...], hbe1[...], mbe1[...]], axis=1)
    s1 = g1 * _BN_SCALE
    W1 = jnp.concatenate(
        [cW1[...], sW1[...], hW1[...], mW1[...]], axis=0)
    W1 = (W1 * s1[0][:, None]).astype(jnp.bfloat16)
    bias1 = s1 * b1 + be1

    def prep2(W2, b2, g2, be2):
        s = g2[...] * _BN_SCALE
        return (W2[...] * s[0][:, None]).astype(jnp.bfloat16), \
            s * b2[...] + be2[...]

    W2s = [prep2(cW2, cb2, cg2, cbe2), prep2(sW2, sb2, sg2, sbe2),
           prep2(hW2, hb2, hg2, hbe2), prep2(mW2, mb2, mg2, mbe2)]

    def padw(W, row_lo, n_rows):
        return jnp.pad(W[...],
                       ((row_lo, _OUT_PAD - row_lo - n_rows), (0, 0))
                       ).astype(jnp.bfloat16)

    Wf = [padw(Wc, 0, 3), padw(Ws, 3, 3), padw(Wh, 6, 2), padw(W3, 8, 20)]
    bias_f = jnp.concatenate([bc[...], bs[...], bh[...], b3[...]], axis=1)
    bias_f = jnp.pad(bias_f, ((0, 0), (0, _OUT_PAD - 28)))

    h1s = []
    for b in range(B):
        pltpu.make_async_copy(x_hbm.at[b], xbuf.at[b], sem.at[b]).wait()
        x = xbuf[b].astype(jnp.bfloat16)
        h1 = jax.lax.dot_general(x, W1, _XT_W,
                                 preferred_element_type=jnp.float32)
        h1s.append(jnp.maximum(h1 + bias1, 0.0).astype(jnp.bfloat16))

    outs = [jnp.pad(xyz_ref[b], ((0, 0), (0, _OUT_PAD - 3))) + bias_f
            for b in range(B)]
    for i in range(4):
        W2, bias2 = W2s[i]
        for b in range(B):
            h2 = jax.lax.dot_general(h1s[b][:, i * 128:(i + 1) * 128], W2,
                                     _HT_W, preferred_element_type=jnp.float32)
            h2 = jnp.maximum(h2 + bias2, 0.0).astype(jnp.bfloat16)
            outs[b] = outs[b] + jax.lax.dot_general(
                h2, Wf[i], _HT_W, preferred_element_type=jnp.float32)
            if i == 3:
                obuf[b] = outs[b][:, :28]
                pltpu.make_async_copy(obuf.at[b], out_ref.at[b],
                                      osem.at[b]).start()
    for b in range(B):
        pltpu.make_async_copy(obuf.at[b], out_ref.at[b], osem.at[b]).wait()


def kernel(vote_features, aggregated_vote_xyz, params):
    B, C, K = vote_features.shape

    def head_args(p):
        return [p['W1'], p['b1'][None, :], p['g1'][None, :], p['be1'][None, :],
                p['W2'], p['b2'][None, :], p['g2'][None, :], p['be2'][None, :]]

    g = params['gmm']
    ps = params['sem']
    args = ([vote_features, aggregated_vote_xyz]
            + head_args(params['center'])
            + head_args(params['size'])
            + head_args(params['heading'])
            + head_args(params['sem'])
            + [g['Wc'], g['bc'][None, :], g['Ws'], g['bs'][None, :],
               g['Wh'], g['bh'][None, :], ps['W3'], ps['b3'][None, :]])

    in_specs = ([pl.BlockSpec(memory_space=pltpu.MemorySpace.HBM),
                 pl.BlockSpec(memory_space=pltpu.MemorySpace.VMEM)]
                + [pl.BlockSpec(memory_space=pltpu.MemorySpace.VMEM)
                   for _ in args[2:]])

    out = pl.pallas_call(
        _fused_kernel,
        in_specs=in_specs,
        out_specs=pl.BlockSpec(memory_space=pltpu.MemorySpace.HBM),
        out_shape=jax.ShapeDtypeStruct((B, K, 28), jnp.float32),
        scratch_shapes=[
            pltpu.VMEM((B, C, K), jnp.float32),
            pltpu.VMEM((B, K, 28), jnp.float32),
            pltpu.SemaphoreType.DMA((B,)),
            pltpu.SemaphoreType.DMA((B,)),
        ],
    )(*args)
    return out
